# Initial kernel scaffold; baseline (speedup 1.0000x reference)
import jax
import jax.numpy as jnp
from jax.experimental import pallas as pl


def kernel(input, embeddings):
    raise NotImplementedError("write your pallas kernel here")


# speedup vs baseline: 5.2675x; 5.2675x over previous
import functools

import jax
import jax.numpy as jnp
from jax import lax
from jax.experimental import pallas as pl
from jax.experimental.pallas import tpu as pltpu
from jax.experimental.pallas import tpu_sc as plsc

_SOM = 8
_K = 64
_LR = 0.05


def _tc_body(B, D, x_ref, en_ref, e_ref, nmin_ref, nemb_ref, com_ref,
             som_ref, t_ref, c_ref, cm_ref, sm_ref):
    i = pl.program_id(0)
    nb = pl.num_programs(0)

    @pl.when(i == 0)
    def _init():
        t_ref[...] = jnp.zeros_like(t_ref)
        c_ref[...] = jnp.zeros_like(c_ref)
        cm_ref[...] = jnp.zeros_like(cm_ref)
        sm_ref[...] = jnp.zeros_like(sm_ref)

    x = x_ref[...]
    e = e_ref[...]
    en = en_ref[...]
    bb = x.shape[0]

    s = lax.dot_general(x, e, (((1,), (1,)), ((), ())),
                        preferred_element_type=jnp.float32)
    xnorm = jnp.sum(x * x, axis=1, keepdims=True)
    dist = (xnorm + en) - 2.0 * s

    mg = jnp.min(dist, axis=1, keepdims=True)
    kio = lax.broadcasted_iota(jnp.int32, dist.shape, 1)
    nm = jnp.min(jnp.where(dist == mg, kio, _K), axis=1,
                 keepdims=True)
    nmin_ref[...] = nm

    nx = lax.shift_right_logical(nm, 3)
    ny = jnp.bitwise_and(nm, 7)
    up = jnp.where(nx < _SOM - 1, nm + _SOM, nm)
    dn = jnp.where(nx > 0, nm - _SOM, nm)
    rt = jnp.where(ny < _SOM - 1, nm + 1, nm)
    lt = jnp.where(ny > 0, nm - 1, nm)

    one = jnp.float32(1.0)
    zero = jnp.float32(0.0)
    oh = jnp.where(kio == nm, one, zero)
    nb4 = (jnp.where(kio == up, one, zero)
           + jnp.where(kio == dn, one, zero)
           + jnp.where(kio == lt, one, zero)
           + jnp.where(kio == rt, one, zero))

    t_ref[...] += lax.dot_general(oh, x, (((0,), (0,)), ((), ())),
                                  preferred_element_type=jnp.float32)
    c_ref[...] += lax.dot_general(oh, jnp.ones((bb, 128), jnp.float32),
                                  (((0,), (0,)), ((), ())),
                                  preferred_element_type=jnp.float32)

    cm_ref[...] += jnp.sum(mg, axis=0, keepdims=True)
    som_rows = jnp.sum(dist * nb4, axis=1, keepdims=True)
    sm_ref[...] += jnp.sum(som_rows, axis=0, keepdims=True)

    @pl.when(i == nb - 1)
    def _fin():
        e_ = e_ref[...]
        c256 = jnp.concatenate([c_ref[...], c_ref[...]], axis=1)
        sseg = t_ref[...] - c256 * e_
        kk = lax.broadcasted_iota(jnp.int32, sseg.shape, 0)
        cc = jnp.bitwise_and(kk, 7)
        z8 = jnp.zeros((8, D), jnp.float32)
        z1 = jnp.zeros((1, D), jnp.float32)
        up_c = (jnp.concatenate([z8, sseg[:-8]], axis=0)
                + jnp.where(kk >= _K - 8, sseg, zero))
        dn_c = (jnp.concatenate([sseg[8:], z8], axis=0)
                + jnp.where(kk < 8, sseg, zero))
        lt_c = (jnp.where(cc <= _SOM - 2,
                          jnp.concatenate([sseg[1:], z1], axis=0), zero)
                + jnp.where(cc == 0, sseg, zero))
        rt_c = (jnp.where(cc >= 1,
                          jnp.concatenate([z1, sseg[:-1]], axis=0), zero)
                + jnp.where(cc == _SOM - 1, sseg, zero))
        nemb_ref[...] = (e_ + _LR * sseg
                         + (0.5 * _LR) * (up_c + dn_c + lt_c + rt_c))
        com_ref[...] = cm_ref[...] * (1.0 / (B * D))
        som_ref[...] = sm_ref[...] * (1.0 / (4 * B * D))


def _tc_part(x, e):
    B, D = x.shape
    K = e.shape[0]
    BB = 2048
    nb = B // BB
    en = jnp.sum(e * e, axis=1)[None, :]
    body = functools.partial(_tc_body, B, D)
    return pl.pallas_call(
        body,
        grid=(nb,),
        in_specs=[
            pl.BlockSpec((BB, D), lambda i: (i, 0)),
            pl.BlockSpec((1, K), lambda i: (0, 0)),
            pl.BlockSpec((K, D), lambda i: (0, 0)),
        ],
        out_specs=[
            pl.BlockSpec((BB, 1), lambda i: (i, 0)),
            pl.BlockSpec((K, D), lambda i: (0, 0)),
            pl.BlockSpec((1, 1), lambda i: (0, 0)),
            pl.BlockSpec((1, 1), lambda i: (0, 0)),
        ],
        out_shape=[
            jax.ShapeDtypeStruct((B, 1), jnp.int32),
            jax.ShapeDtypeStruct((K, D), jnp.float32),
            jax.ShapeDtypeStruct((1, 1), jnp.float32),
            jax.ShapeDtypeStruct((1, 1), jnp.float32),
        ],
        scratch_shapes=[
            pltpu.VMEM((K, D), jnp.float32),
            pltpu.VMEM((K, 128), jnp.float32),
            pltpu.VMEM((1, 1), jnp.float32),
            pltpu.VMEM((1, 1), jnp.float32),
        ],
    )(x, en, e)


def _sc_gather(emb, idx):
    B = idx.shape[0]
    D = emb.shape[1]
    info = plsc.get_sparse_core_info()
    ncores, nsub = info.num_cores, info.num_subcores
    nw = ncores * nsub
    bpw = B // nw
    ch = 128
    nch = bpw // ch
    mesh = plsc.VectorSubcoreMesh(core_axis_name="c", subcore_axis_name="s")

    @functools.partial(
        pl.kernel, mesh=mesh,
        out_type=jax.ShapeDtypeStruct((B, D), jnp.float32),
        scratch_types=[
            pltpu.VMEM((ch,), jnp.int32),
            pltpu.VMEM((ch, D), jnp.float32),
            pltpu.SemaphoreType.DMA,
        ],
    )
    def gk(table_hbm, idx_hbm, out_hbm, idx_v, rows_v, sem):
        wid = lax.axis_index("s") * ncores + lax.axis_index("c")
        base = wid * bpw
        for j in range(nch):
            off = base + j * ch
            pltpu.sync_copy(idx_hbm.at[pl.ds(off, ch)], idx_v)
            pltpu.async_copy(table_hbm.at[idx_v], rows_v, sem).wait()
            pltpu.sync_copy(rows_v, out_hbm.at[pl.ds(off, ch)])

    return gk(emb, idx)


def kernel(input, embeddings):
    nmin2, new_emb, com, som = _tc_part(input, embeddings)
    z_q = _sc_gather(embeddings, nmin2.reshape(-1))
    return z_q, com.reshape(()), som.reshape(()), new_emb


# speedup vs baseline: 5.3244x; 1.0108x over previous
import functools

import jax
import jax.numpy as jnp
from jax import lax
from jax.experimental import pallas as pl
from jax.experimental.pallas import tpu as pltpu
from jax.experimental.pallas import tpu_sc as plsc

_SOM = 8
_K = 64
_LR = 0.05


def _tc_body(B, D, x_ref, en_ref, e_ref, nmin_ref, nemb_ref, com_ref,
             som_ref, t_ref, c_ref, cm_ref, sm_ref):
    i = pl.program_id(0)
    nb = pl.num_programs(0)

    @pl.when(i == 0)
    def _init():
        t_ref[...] = jnp.zeros_like(t_ref)
        c_ref[...] = jnp.zeros_like(c_ref)
        cm_ref[...] = jnp.zeros_like(cm_ref)
        sm_ref[...] = jnp.zeros_like(sm_ref)

    x = x_ref[...]
    e = e_ref[...]
    en = en_ref[...]
    bb = x.shape[0]

    s = lax.dot_general(x, e, (((1,), (1,)), ((), ())),
                        preferred_element_type=jnp.float32)
    xnorm = jnp.sum(x * x, axis=1, keepdims=True)
    dist = (xnorm + en) - 2.0 * s

    mg = jnp.min(dist, axis=1, keepdims=True)
    kio = lax.broadcasted_iota(jnp.int32, dist.shape, 1)
    nm = jnp.min(jnp.where(dist == mg, kio, _K), axis=1,
                 keepdims=True)
    nmin_ref[...] = nm

    nx = lax.shift_right_logical(nm, 3)
    ny = jnp.bitwise_and(nm, 7)
    up = jnp.where(nx < _SOM - 1, nm + _SOM, nm)
    dn = jnp.where(nx > 0, nm - _SOM, nm)
    rt = jnp.where(ny < _SOM - 1, nm + 1, nm)
    lt = jnp.where(ny > 0, nm - 1, nm)

    one = jnp.float32(1.0)
    zero = jnp.float32(0.0)
    oh = jnp.where(kio == nm, one, zero)
    nb4 = (jnp.where(kio == up, one, zero)
           + jnp.where(kio == dn, one, zero)
           + jnp.where(kio == lt, one, zero)
           + jnp.where(kio == rt, one, zero))

    t_ref[...] += lax.dot_general(oh, x, (((0,), (0,)), ((), ())),
                                  preferred_element_type=jnp.float32)
    c_ref[...] += lax.dot_general(oh, jnp.ones((bb, 128), jnp.float32),
                                  (((0,), (0,)), ((), ())),
                                  preferred_element_type=jnp.float32)

    cm_ref[...] += jnp.sum(mg, axis=0, keepdims=True)
    som_rows = jnp.sum(dist * nb4, axis=1, keepdims=True)
    sm_ref[...] += jnp.sum(som_rows, axis=0, keepdims=True)

    @pl.when(i == nb - 1)
    def _fin():
        e_ = e_ref[...]
        c256 = jnp.concatenate([c_ref[...], c_ref[...]], axis=1)
        sseg = t_ref[...] - c256 * e_
        kk = lax.broadcasted_iota(jnp.int32, sseg.shape, 0)
        cc = jnp.bitwise_and(kk, 7)
        z8 = jnp.zeros((8, D), jnp.float32)
        z1 = jnp.zeros((1, D), jnp.float32)
        up_c = (jnp.concatenate([z8, sseg[:-8]], axis=0)
                + jnp.where(kk >= _K - 8, sseg, zero))
        dn_c = (jnp.concatenate([sseg[8:], z8], axis=0)
                + jnp.where(kk < 8, sseg, zero))
        lt_c = (jnp.where(cc <= _SOM - 2,
                          jnp.concatenate([sseg[1:], z1], axis=0), zero)
                + jnp.where(cc == 0, sseg, zero))
        rt_c = (jnp.where(cc >= 1,
                          jnp.concatenate([z1, sseg[:-1]], axis=0), zero)
                + jnp.where(cc == _SOM - 1, sseg, zero))
        nemb_ref[...] = (e_ + _LR * sseg
                         + (0.5 * _LR) * (up_c + dn_c + lt_c + rt_c))
        com_ref[...] = cm_ref[...] * (1.0 / (B * D))
        som_ref[...] = sm_ref[...] * (1.0 / (4 * B * D))


def _tc_part(x, e):
    B, D = x.shape
    K = e.shape[0]
    BB = 2048
    nb = B // BB
    en = jnp.sum(e * e, axis=1)[None, :]
    body = functools.partial(_tc_body, B, D)
    return pl.pallas_call(
        body,
        grid=(nb,),
        in_specs=[
            pl.BlockSpec((BB, D), lambda i: (i, 0)),
            pl.BlockSpec((1, K), lambda i: (0, 0)),
            pl.BlockSpec((K, D), lambda i: (0, 0)),
        ],
        out_specs=[
            pl.BlockSpec((BB, 1), lambda i: (i, 0)),
            pl.BlockSpec((K, D), lambda i: (0, 0)),
            pl.BlockSpec((1, 1), lambda i: (0, 0)),
            pl.BlockSpec((1, 1), lambda i: (0, 0)),
        ],
        out_shape=[
            jax.ShapeDtypeStruct((B, 1), jnp.int32),
            jax.ShapeDtypeStruct((K, D), jnp.float32),
            jax.ShapeDtypeStruct((1, 1), jnp.float32),
            jax.ShapeDtypeStruct((1, 1), jnp.float32),
        ],
        scratch_shapes=[
            pltpu.VMEM((K, D), jnp.float32),
            pltpu.VMEM((K, 128), jnp.float32),
            pltpu.VMEM((1, 1), jnp.float32),
            pltpu.VMEM((1, 1), jnp.float32),
        ],
    )(x, en, e)


def _sc_gather(emb, idx):
    B = idx.shape[0]
    D = emb.shape[1]
    info = plsc.get_sparse_core_info()
    ncores, nsub = info.num_cores, info.num_subcores
    nw = ncores * nsub
    bpw = B // nw
    ch = 128
    nch = bpw // ch
    mesh = plsc.VectorSubcoreMesh(core_axis_name="c", subcore_axis_name="s")

    @functools.partial(
        pl.kernel, mesh=mesh,
        out_type=jax.ShapeDtypeStruct((B, D), jnp.float32),
        scratch_types=[
            pltpu.VMEM((nch, ch), jnp.int32),
            pltpu.VMEM((ch, D), jnp.float32),
            pltpu.VMEM((ch, D), jnp.float32),
            pltpu.SemaphoreType.DMA,
            pltpu.SemaphoreType.DMA,
            pltpu.SemaphoreType.DMA,
            pltpu.SemaphoreType.DMA,
        ],
    )
    def gk(table_hbm, idx_hbm, out_hbm, idx_v, r0, r1, g0, g1, o0, o1):
        wid = lax.axis_index("s") * ncores + lax.axis_index("c")
        base = wid * bpw
        bufs, gsem, osem = [r0, r1], [g0, g1], [o0, o1]
        pltpu.sync_copy(idx_hbm.at[pl.ds(wid * nch, nch)], idx_v)
        gd = [None] * nch
        od = [None] * nch
        gd[0] = pltpu.async_copy(table_hbm.at[idx_v.at[0]], bufs[0], gsem[0])
        for j in range(nch):
            b = j & 1
            if j + 1 < nch:
                if j >= 1:
                    od[j - 1].wait()
                gd[j + 1] = pltpu.async_copy(table_hbm.at[idx_v.at[j + 1]],
                                             bufs[1 - b], gsem[1 - b])
            gd[j].wait()
            od[j] = pltpu.async_copy(bufs[b],
                                     out_hbm.at[pl.ds(base + j * ch, ch)],
                                     osem[b])
        od[nch - 2].wait()
        od[nch - 1].wait()

    return gk(emb, idx.reshape(nw * nch, ch))


def kernel(input, embeddings):
    nmin2, new_emb, com, som = _tc_part(input, embeddings)
    z_q = _sc_gather(embeddings, nmin2.reshape(-1))
    return z_q, com.reshape(()), som.reshape(()), new_emb


# speedup vs baseline: 6.8849x; 1.2931x over previous
import functools

import jax
import jax.numpy as jnp
from jax import lax
from jax.experimental import pallas as pl
from jax.experimental.pallas import tpu as pltpu
from jax.experimental.pallas import tpu_sc as plsc

_SOM = 8
_K = 64
_LR = 0.05


def _tc_body(B, D, x_ref, en_ref, e_ref, nmin_ref, nemb_ref, com_ref,
             som_ref, t_ref, c_ref, cm_ref, sm_ref):
    i = pl.program_id(0)
    nb = pl.num_programs(0)

    @pl.when(i == 0)
    def _init():
        t_ref[...] = jnp.zeros_like(t_ref)
        c_ref[...] = jnp.zeros_like(c_ref)
        cm_ref[...] = jnp.zeros_like(cm_ref)
        sm_ref[...] = jnp.zeros_like(sm_ref)

    x = x_ref[...]
    e = e_ref[...]
    en = en_ref[...]
    bb = x.shape[0]

    s = lax.dot_general(x, e, (((1,), (1,)), ((), ())),
                        preferred_element_type=jnp.float32)
    xnorm = jnp.sum(x * x, axis=1, keepdims=True)
    dist = (xnorm + en) - 2.0 * s

    mg = jnp.min(dist, axis=1, keepdims=True)
    kio = lax.broadcasted_iota(jnp.int32, dist.shape, 1)
    nm = jnp.min(jnp.where(dist == mg, kio, _K), axis=1,
                 keepdims=True)
    nmin_ref[...] = nm

    nx = lax.shift_right_logical(nm, 3)
    ny = jnp.bitwise_and(nm, 7)
    up = jnp.where(nx < _SOM - 1, nm + _SOM, nm)
    dn = jnp.where(nx > 0, nm - _SOM, nm)
    rt = jnp.where(ny < _SOM - 1, nm + 1, nm)
    lt = jnp.where(ny > 0, nm - 1, nm)

    one = jnp.float32(1.0)
    zero = jnp.float32(0.0)
    oh = jnp.where(kio == nm, one, zero)
    nb4 = (jnp.where(kio == up, one, zero)
           + jnp.where(kio == dn, one, zero)
           + jnp.where(kio == lt, one, zero)
           + jnp.where(kio == rt, one, zero))

    t_ref[...] += lax.dot_general(oh, x, (((0,), (0,)), ((), ())),
                                  preferred_element_type=jnp.float32)
    c_ref[...] += lax.dot_general(oh, jnp.ones((bb, 128), jnp.float32),
                                  (((0,), (0,)), ((), ())),
                                  preferred_element_type=jnp.float32)

    cm_ref[...] += jnp.sum(mg, axis=0, keepdims=True)
    som_rows = jnp.sum(dist * nb4, axis=1, keepdims=True)
    sm_ref[...] += jnp.sum(som_rows, axis=0, keepdims=True)

    @pl.when(i == nb - 1)
    def _fin():
        e_ = e_ref[...]
        c256 = jnp.concatenate([c_ref[...], c_ref[...]], axis=1)
        sseg = t_ref[...] - c256 * e_
        kk = lax.broadcasted_iota(jnp.int32, sseg.shape, 0)
        cc = jnp.bitwise_and(kk, 7)
        z8 = jnp.zeros((8, D), jnp.float32)
        z1 = jnp.zeros((1, D), jnp.float32)
        up_c = (jnp.concatenate([z8, sseg[:-8]], axis=0)
                + jnp.where(kk >= _K - 8, sseg, zero))
        dn_c = (jnp.concatenate([sseg[8:], z8], axis=0)
                + jnp.where(kk < 8, sseg, zero))
        lt_c = (jnp.where(cc <= _SOM - 2,
                          jnp.concatenate([sseg[1:], z1], axis=0), zero)
                + jnp.where(cc == 0, sseg, zero))
        rt_c = (jnp.where(cc >= 1,
                          jnp.concatenate([z1, sseg[:-1]], axis=0), zero)
                + jnp.where(cc == _SOM - 1, sseg, zero))
        nemb_ref[...] = (e_ + _LR * sseg
                         + (0.5 * _LR) * (up_c + dn_c + lt_c + rt_c))
        com_ref[...] = cm_ref[...] * (1.0 / (B * D))
        som_ref[...] = sm_ref[...] * (1.0 / (4 * B * D))


def _tc_part(x, e):
    B, D = x.shape
    K = e.shape[0]
    BB = 2048
    nb = B // BB
    en = jnp.sum(e * e, axis=1)[None, :]
    body = functools.partial(_tc_body, B, D)
    return pl.pallas_call(
        body,
        grid=(nb,),
        in_specs=[
            pl.BlockSpec((BB, D), lambda i: (i, 0)),
            pl.BlockSpec((1, K), lambda i: (0, 0)),
            pl.BlockSpec((K, D), lambda i: (0, 0)),
        ],
        out_specs=[
            pl.BlockSpec((BB, 1), lambda i: (i, 0)),
            pl.BlockSpec((K, D), lambda i: (0, 0)),
            pl.BlockSpec((1, 1), lambda i: (0, 0)),
            pl.BlockSpec((1, 1), lambda i: (0, 0)),
        ],
        out_shape=[
            jax.ShapeDtypeStruct((B, 1), jnp.int32),
            jax.ShapeDtypeStruct((K, D), jnp.float32),
            jax.ShapeDtypeStruct((1, 1), jnp.float32),
            jax.ShapeDtypeStruct((1, 1), jnp.float32),
        ],
        scratch_shapes=[
            pltpu.VMEM((K, D), jnp.float32),
            pltpu.VMEM((K, 128), jnp.float32),
            pltpu.VMEM((1, 1), jnp.float32),
            pltpu.VMEM((1, 1), jnp.float32),
        ],
    )(x, en, e)


def _sc_gather(emb, idx):
    B = idx.shape[0]
    K, D = emb.shape
    info = plsc.get_sparse_core_info()
    ncores, nsub = info.num_cores, info.num_subcores
    nw = ncores * nsub
    bpw = B // nw
    ch = 128
    nch = bpw // ch
    mesh = plsc.VectorSubcoreMesh(core_axis_name="c", subcore_axis_name="s")

    @functools.partial(
        pl.kernel, mesh=mesh,
        out_type=jax.ShapeDtypeStruct((B * D,), jnp.float32),
        scratch_types=[
            pltpu.VMEM((K * D,), jnp.float32),
            pltpu.VMEM((bpw,), jnp.int32),
            pltpu.VMEM((ch * D,), jnp.float32),
            pltpu.VMEM((ch * D,), jnp.float32),
            pltpu.SemaphoreType.DMA,
            pltpu.SemaphoreType.DMA,
        ],
    )
    def gk(table_hbm, idx_hbm, out_hbm, tab_v, idx_v, b0, b1, o0, o1):
        wid = lax.axis_index("s") * ncores + lax.axis_index("c")
        base = wid * bpw
        bufs, osem = [b0, b1], [o0, o1]
        pltpu.sync_copy(idx_hbm.at[pl.ds(base, bpw)], idx_v)
        pltpu.sync_copy(table_hbm, tab_v)
        od = [None, None]
        for j in range(nch):
            b = j & 1
            if od[b] is not None:
                od[b].wait()

            def grp(g, _, j=j, b=b):
                kv = idx_v[pl.ds(j * ch + g * 16, 16)]
                for u in range(16):
                    src = kv[u] * D
                    dst = (g * 16 + u) * D
                    for c in range(D // 16):
                        bufs[b][pl.ds(dst + c * 16, 16)] = (
                            tab_v[pl.ds(src + c * 16, 16)])
                return 0

            lax.fori_loop(0, ch // 16, grp, 0)
            od[b] = pltpu.async_copy(
                bufs[b], out_hbm.at[pl.ds((base + j * ch) * D, ch * D)],
                osem[b])
        od[0].wait()
        od[1].wait()

    return gk(emb.reshape(-1), idx).reshape(B, D)


def kernel(input, embeddings):
    nmin2, new_emb, com, som = _tc_part(input, embeddings)
    z_q = _sc_gather(embeddings, nmin2.reshape(-1))
    return z_q, com.reshape(()), som.reshape(()), new_emb


# speedup vs baseline: 7.1957x; 1.0451x over previous
import functools

import jax
import jax.numpy as jnp
from jax import lax
from jax.experimental import pallas as pl
from jax.experimental.pallas import tpu as pltpu
from jax.experimental.pallas import tpu_sc as plsc

_SOM = 8
_K = 64
_LR = 0.05


def _tc_body(B, D, x_ref, e_ref, en_ref, nmin_ref, nemb_ref, com_ref,
             som_ref, t_ref, c_ref, cm_ref, sm_ref, u_ref, v_ref):
    i = pl.program_id(0)
    nb = pl.num_programs(0)
    e = e_ref[...]

    @pl.when(i == 0)
    def _init():
        t_ref[...] = jnp.zeros_like(t_ref)
        c_ref[...] = jnp.zeros_like(c_ref)
        cm_ref[...] = jnp.zeros_like(cm_ref)
        sm_ref[...] = jnp.zeros_like(sm_ref)
        bbs = u_ref.shape[0]
        bu = lax.broadcasted_iota(jnp.int32, u_ref.shape, 0)
        gu = lax.broadcasted_iota(jnp.int32, u_ref.shape, 1)
        u_ref[...] = jnp.where(lax.shift_right_logical(bu, 7) == gu,
                               jnp.float32(1.0), jnp.float32(0.0))
        bv = lax.broadcasted_iota(jnp.int32, v_ref.shape, 0)
        jv = lax.broadcasted_iota(jnp.int32, v_ref.shape, 1)
        v_ref[...] = jnp.where(jnp.bitwise_and(bv, 127) == jv,
                               jnp.float32(1.0), jnp.float32(0.0))

    x = x_ref[...]
    en = en_ref[...]
    bb = x.shape[0]

    s = lax.dot_general(x, e, (((1,), (1,)), ((), ())),
                        preferred_element_type=jnp.float32)
    xnorm = jnp.sum(x * x, axis=1, keepdims=True)
    dist = (xnorm + en) - 2.0 * s

    mg = jnp.min(dist, axis=1, keepdims=True)
    kio = lax.broadcasted_iota(jnp.int32, dist.shape, 1)
    nm = jnp.min(jnp.where(dist == mg, kio, _K), axis=1,
                 keepdims=True)
    nm2d = lax.dot_general(u_ref[...] * nm.astype(jnp.float32), v_ref[...],
                           (((0,), (0,)), ((), ())),
                           preferred_element_type=jnp.float32)
    nmin_ref[...] = nm2d.astype(jnp.int32)

    nx = lax.shift_right_logical(nm, 3)
    ny = jnp.bitwise_and(nm, 7)
    up = jnp.where(nx < _SOM - 1, nm + _SOM, nm)
    dn = jnp.where(nx > 0, nm - _SOM, nm)
    rt = jnp.where(ny < _SOM - 1, nm + 1, nm)
    lt = jnp.where(ny > 0, nm - 1, nm)

    one = jnp.float32(1.0)
    zero = jnp.float32(0.0)
    oh = jnp.where(kio == nm, one, zero)
    nb4 = (jnp.where(kio == up, one, zero)
           + jnp.where(kio == dn, one, zero)
           + jnp.where(kio == lt, one, zero)
           + jnp.where(kio == rt, one, zero))

    t_ref[...] += lax.dot_general(oh, x, (((0,), (0,)), ((), ())),
                                  preferred_element_type=jnp.float32)
    c_ref[...] += lax.dot_general(oh, jnp.ones((bb, 128), jnp.float32),
                                  (((0,), (0,)), ((), ())),
                                  preferred_element_type=jnp.float32)

    cm_ref[...] += jnp.sum(mg, axis=0, keepdims=True)
    som_rows = jnp.sum(dist * nb4, axis=1, keepdims=True)
    sm_ref[...] += jnp.sum(som_rows, axis=0, keepdims=True)

    @pl.when(i == nb - 1)
    def _fin():
        e_ = e_ref[...]
        c256 = jnp.concatenate([c_ref[...], c_ref[...]], axis=1)
        sseg = t_ref[...] - c256 * e_
        kk = lax.broadcasted_iota(jnp.int32, sseg.shape, 0)
        cc = jnp.bitwise_and(kk, 7)
        z8 = jnp.zeros((8, D), jnp.float32)
        z1 = jnp.zeros((1, D), jnp.float32)
        up_c = (jnp.concatenate([z8, sseg[:-8]], axis=0)
                + jnp.where(kk >= _K - 8, sseg, zero))
        dn_c = (jnp.concatenate([sseg[8:], z8], axis=0)
                + jnp.where(kk < 8, sseg, zero))
        lt_c = (jnp.where(cc <= _SOM - 2,
                          jnp.concatenate([sseg[1:], z1], axis=0), zero)
                + jnp.where(cc == 0, sseg, zero))
        rt_c = (jnp.where(cc >= 1,
                          jnp.concatenate([z1, sseg[:-1]], axis=0), zero)
                + jnp.where(cc == _SOM - 1, sseg, zero))
        nemb_ref[...] = (e_ + _LR * sseg
                         + (0.5 * _LR) * (up_c + dn_c + lt_c + rt_c))
        com_ref[...] = cm_ref[...] * (1.0 / (B * D))
        som_ref[...] = sm_ref[...] * (1.0 / (4 * B * D))


def _tc_part(x, e, en):
    B, D = x.shape
    K = e.shape[0]
    BB = 2048
    nb = B // BB
    body = functools.partial(_tc_body, B, D)
    return pl.pallas_call(
        body,
        grid=(nb,),
        in_specs=[
            pl.BlockSpec((BB, D), lambda i: (i, 0)),
            pl.BlockSpec((K, D), lambda i: (0, 0)),
            pl.BlockSpec((1, K), lambda i: (0, 0)),
        ],
        out_specs=[
            pl.BlockSpec((BB // 128, 128), lambda i: (i, 0)),
            pl.BlockSpec((K, D), lambda i: (0, 0)),
            pl.BlockSpec((1, 1), lambda i: (0, 0)),
            pl.BlockSpec((1, 1), lambda i: (0, 0)),
        ],
        out_shape=[
            jax.ShapeDtypeStruct((B // 128, 128), jnp.int32),
            jax.ShapeDtypeStruct((K, D), jnp.float32),
            jax.ShapeDtypeStruct((1, 1), jnp.float32),
            jax.ShapeDtypeStruct((1, 1), jnp.float32),
        ],
        scratch_shapes=[
            pltpu.VMEM((K, D), jnp.float32),
            pltpu.VMEM((K, 128), jnp.float32),
            pltpu.VMEM((1, 1), jnp.float32),
            pltpu.VMEM((1, 1), jnp.float32),
            pltpu.VMEM((BB, 16), jnp.float32),
            pltpu.VMEM((BB, 128), jnp.float32),
        ],
    )(x, e, en)


def _sc_gather(emb, idx):
    B = idx.shape[0]
    K, D = emb.shape
    info = plsc.get_sparse_core_info()
    ncores, nsub = info.num_cores, info.num_subcores
    nw = ncores * nsub
    bpw = B // nw
    ch = 128
    nch = bpw // ch
    mesh = plsc.VectorSubcoreMesh(core_axis_name="c", subcore_axis_name="s")

    @functools.partial(
        pl.kernel, mesh=mesh,
        out_type=jax.ShapeDtypeStruct((B * D,), jnp.float32),
        scratch_types=[
            pltpu.VMEM((K * D,), jnp.float32),
            pltpu.VMEM((bpw,), jnp.int32),
            pltpu.VMEM((ch * D,), jnp.float32),
            pltpu.VMEM((ch * D,), jnp.float32),
            pltpu.SemaphoreType.DMA,
            pltpu.SemaphoreType.DMA,
        ],
    )
    def gk(table_hbm, idx_hbm, out_hbm, tab_v, idx_v, b0, b1, o0, o1):
        wid = lax.axis_index("s") * ncores + lax.axis_index("c")
        base = wid * bpw
        bufs, osem = [b0, b1], [o0, o1]
        pltpu.sync_copy(idx_hbm.at[pl.ds(base, bpw)], idx_v)
        pltpu.sync_copy(table_hbm, tab_v)
        od = [None, None]
        for j in range(nch):
            b = j & 1
            if od[b] is not None:
                od[b].wait()

            def grp(g, _, j=j, b=b):
                kv = idx_v[pl.ds(j * ch + g * 16, 16)]
                for u in range(16):
                    src = kv[u] * D
                    dst = (g * 16 + u) * D
                    for c in range(D // 16):
                        bufs[b][pl.ds(dst + c * 16, 16)] = (
                            tab_v[pl.ds(src + c * 16, 16)])
                return 0

            lax.fori_loop(0, ch // 16, grp, 0)
            od[b] = pltpu.async_copy(
                bufs[b], out_hbm.at[pl.ds((base + j * ch) * D, ch * D)],
                osem[b])
        od[0].wait()
        od[1].wait()

    return gk(emb.reshape(-1), idx).reshape(B, D)


def kernel(input, embeddings):
    en = jnp.sum(embeddings ** 2, axis=1)[None, :]
    nmin2, new_emb, com, som = _tc_part(input, embeddings, en)
    z_q = _sc_gather(embeddings, nmin2.reshape(-1))
    return z_q, com.reshape(()), som.reshape(()), new_emb
